# Initial kernel scaffold; baseline (speedup 1.0000x reference)
import jax
import jax.numpy as jnp
from jax.experimental import pallas as pl


def kernel(poi_counts, table):
    raise NotImplementedError("write your pallas kernel here")


# speedup vs baseline: 3.7944x; 3.7944x over previous
import functools

import jax
import jax.numpy as jnp
from jax import lax
from jax.experimental import pallas as pl
from jax.experimental.pallas import tpu as pltpu
from jax.experimental.pallas import tpu_sc as plsc

_NUM_EMB = 736
_D = 64
_IDX_W = 128


@functools.partial(jax.jit, static_argnums=(2, 3))
def _sc_embedding_gather(idx2d, table, B, C):
    NW = 32
    K = C // _IDX_W
    b_per_w = B // NW
    n_chunks = b_per_w // C
    mesh = plsc.VectorSubcoreMesh(core_axis_name="c", subcore_axis_name="s")

    @functools.partial(
        pl.kernel,
        mesh=mesh,
        out_type=jax.ShapeDtypeStruct((B, _D), jnp.float32),
        scratch_types=[
            pltpu.VMEM((K, _IDX_W), jnp.int32),
            pltpu.VMEM((C, _D), jnp.float32),
            pltpu.SemaphoreType.DMA,
        ],
        compiler_params=pltpu.CompilerParams(use_tc_tiling_on_sc=False),
    )
    def k(table_hbm, idx_hbm, out_hbm, idx_v, rows_v, sem):
        wid = lax.axis_index("s") * 2 + lax.axis_index("c")
        base = wid * b_per_w

        def body(c, carry):
            start = pl.multiple_of(base + c * C, 512)
            irow = pl.multiple_of(base // _IDX_W + c * K, 8)
            pltpu.sync_copy(idx_hbm.at[pl.ds(irow, K)], idx_v)
            copies = [
                pltpu.async_copy(
                    table_hbm.at[idx_v.at[j]],
                    rows_v.at[pl.ds(j * _IDX_W, _IDX_W)],
                    sem,
                )
                for j in range(K)
            ]
            for cp in copies:
                cp.wait()
            pltpu.sync_copy(rows_v, out_hbm.at[pl.ds(start, C)])
            return carry

        lax.fori_loop(0, n_chunks, body, 0)

    return k(table, idx2d)


def kernel(poi_counts, table):
    n, m = poi_counts.shape
    B = n * m
    idx2d = poi_counts.reshape(B // _IDX_W, _IDX_W)
    flat = _sc_embedding_gather(idx2d, table, B, 1024)
    return flat.reshape(n, m, _D)


# speedup vs baseline: 3.8248x; 1.0080x over previous
import functools

import jax
import jax.numpy as jnp
from jax import lax
from jax.experimental import pallas as pl
from jax.experimental.pallas import tpu as pltpu
from jax.experimental.pallas import tpu_sc as plsc

_D = 64
_IDX_W = 128


@functools.partial(jax.jit, static_argnums=(2, 3))
def _sc_embedding_gather(idx_flat, table, B, C):
    NW = 32
    K = C // _IDX_W
    b_per_w = B // NW
    n_chunks = b_per_w // C
    assert n_chunks % 2 == 0
    mesh = plsc.VectorSubcoreMesh(core_axis_name="c", subcore_axis_name="s")

    @functools.partial(
        pl.kernel,
        mesh=mesh,
        out_type=jax.ShapeDtypeStruct((B, _D), jnp.float32),
        scratch_types=[
            pltpu.VMEM((2, C), jnp.int32),
            pltpu.VMEM((2, C, _D), jnp.float32),
            pltpu.SemaphoreType.DMA,
            pltpu.SemaphoreType.DMA,
            pltpu.SemaphoreType.DMA,
            pltpu.SemaphoreType.DMA,
        ],
        compiler_params=pltpu.CompilerParams(use_tc_tiling_on_sc=False),
    )
    def k(table_hbm, idx_hbm, out_hbm, idx_v, rows_v, g0, g1, s0, s1):
        sem_g = (g0, g1)
        sem_s = (s0, s1)
        wid = lax.axis_index("s") * 2 + lax.axis_index("c")
        base = wid * b_per_w

        def body(t, carry):
            for b in range(2):
                ch = 2 * t + b
                start = pl.multiple_of(base + ch * C, C)

                @pl.when(t > 0)
                def _drain():
                    pltpu.make_async_copy(
                        rows_v.at[b], out_hbm.at[pl.ds(0, C)], sem_s[b]
                    ).wait()

                pltpu.sync_copy(idx_hbm.at[pl.ds(start, C)], idx_v.at[b])
                gathers = [
                    pltpu.async_copy(
                        table_hbm.at[idx_v.at[b, pl.ds(j * _IDX_W, _IDX_W)]],
                        rows_v.at[b, pl.ds(j * _IDX_W, _IDX_W)],
                        sem_g[b],
                    )
                    for j in range(K)
                ]
                for cp in gathers:
                    cp.wait()
                pltpu.async_copy(
                    rows_v.at[b], out_hbm.at[pl.ds(start, C)], sem_s[b]
                )
            return carry

        lax.fori_loop(0, n_chunks // 2, body, 0)
        for b in range(2):
            pltpu.make_async_copy(
                rows_v.at[b], out_hbm.at[pl.ds(0, C)], sem_s[b]
            ).wait()

    return k(table, idx_flat)


def kernel(poi_counts, table):
    n, m = poi_counts.shape
    B = n * m
    flat = _sc_embedding_gather(poi_counts.reshape(B), table, B, 640)
    return flat.reshape(n, m, _D)


# speedup vs baseline: 7.4030x; 1.9355x over previous
import functools

import jax
import jax.numpy as jnp
from jax import lax
from jax.experimental import pallas as pl
from jax.experimental.pallas import tpu as pltpu
from jax.experimental.pallas import tpu_sc as plsc

_V = 736
_D = 64
_S = 200
_BATCH = 16384


@jax.jit
def _sc_embedding_gather(tab_t_flat, idx_t):
    NW = 32
    per_w = _BATCH // NW
    n_chunks = per_w // 128
    mesh = plsc.VectorSubcoreMesh(core_axis_name="c", subcore_axis_name="s")

    @functools.partial(
        pl.kernel,
        mesh=mesh,
        out_type=jax.ShapeDtypeStruct(
            (_S * _D // 8, _BATCH // 128, 8, 128), jnp.float32
        ),
        scratch_types=[
            pltpu.VMEM((_D * _V,), jnp.float32),
            pltpu.VMEM((_S, 128), jnp.int32),
            pltpu.VMEM((2, 8, 1, 8, 128), jnp.float32),
            pltpu.SemaphoreType.DMA,
            pltpu.SemaphoreType.DMA,
        ],
        compiler_params=pltpu.CompilerParams(
            use_tc_tiling_on_sc=False, needs_layout_passes=False
        ),
    )
    def k(tab_hbm, idx_hbm, out_hbm, table_v, idx_v, out_v, sem0, sem1):
        sem_s = (sem0, sem1)
        wid = lax.axis_index("s") * 2 + lax.axis_index("c")
        pltpu.sync_copy(tab_hbm, table_v)
        for ci in range(n_chunks):
            i0 = pl.multiple_of(wid * per_w + ci * 128, 128)
            it = pl.multiple_of(wid * n_chunks + ci, 1)
            pltpu.sync_copy(idx_hbm.at[:, pl.ds(i0, 128)], idx_v)

            def pair(p, carry):
                for b in range(2):
                    s = 2 * p + b

                    @pl.when(p > 0)
                    def _drain():
                        pltpu.make_async_copy(
                            out_v.at[b],
                            out_hbm.at[pl.ds(0, 8), pl.ds(0, 1), :, :],
                            sem_s[b],
                        ).wait()

                    def group(g, carry2):
                        idx16 = idx_v[s, pl.ds(g * 16, 16)]
                        for d in range(_D):
                            out_v[b, d // 8, 0, d % 8, pl.ds(g * 16, 16)] = (
                                plsc.load_gather(table_v, [idx16 + d * _V])
                            )
                        return carry2

                    lax.fori_loop(0, 8, group, 0)
                    pltpu.async_copy(
                        out_v.at[b],
                        out_hbm.at[
                            pl.ds(pl.multiple_of(s * 8, 8), 8),
                            pl.ds(it, 1),
                            :,
                            :,
                        ],
                        sem_s[b],
                    )
                return carry

            lax.fori_loop(0, _S // 2, pair, 0)
            for b in range(2):
                pltpu.make_async_copy(
                    out_v.at[b],
                    out_hbm.at[pl.ds(0, 8), pl.ds(0, 1), :, :],
                    sem_s[b],
                ).wait()

    return k(tab_t_flat, idx_t)


def kernel(poi_counts, table):
    out4 = _sc_embedding_gather(table.T.reshape(-1), poi_counts.T)
    return (
        out4.reshape(_S, 8, _BATCH // 128, 8, 128)
        .transpose(2, 4, 0, 1, 3)
        .reshape(_BATCH, _S, _D)
    )


# speedup vs baseline: 17.6715x; 2.3871x over previous
import functools

import jax
import jax.numpy as jnp
from jax import lax
from jax.experimental import pallas as pl
from jax.experimental.pallas import tpu as pltpu
from jax.experimental.pallas import tpu_sc as plsc

_V = 736
_D = 64
_S = 200
_BATCH = 16384


@jax.jit
def _sc_embedding_gather(tab_t_flat, idx_t):
    NW = 32
    per_w = _BATCH // NW
    n_chunks = per_w // 128
    mesh = plsc.VectorSubcoreMesh(core_axis_name="c", subcore_axis_name="s")

    @functools.partial(
        pl.kernel,
        mesh=mesh,
        out_type=jax.ShapeDtypeStruct(
            (_S * _D // 8, _BATCH // 128, 8, 128), jnp.float32
        ),
        scratch_types=[
            pltpu.VMEM((_D * _V,), jnp.float32),
            pltpu.VMEM((_S, 128), jnp.int32),
            pltpu.VMEM((2, 8, 1, 8, 128), jnp.float32),
            pltpu.SemaphoreType.DMA,
            pltpu.SemaphoreType.DMA,
        ],
        compiler_params=pltpu.CompilerParams(
            use_tc_tiling_on_sc=False, needs_layout_passes=False
        ),
    )
    def k(tab_hbm, idx_hbm, out_hbm, table_v, idx_v, out_v, sem0, sem1):
        sem_s = (sem0, sem1)
        wid = lax.axis_index("s") * 2 + lax.axis_index("c")
        pltpu.sync_copy(tab_hbm, table_v)
        for ci in range(n_chunks):
            i0 = pl.multiple_of(wid * per_w + ci * 128, 128)
            it = pl.multiple_of(wid * n_chunks + ci, 1)
            pltpu.sync_copy(idx_hbm.at[:, pl.ds(i0, 128)], idx_v)

            def pair(p, carry):
                for b in range(2):
                    s = 2 * p + b

                    @pl.when(p > 0)
                    def _drain():
                        pltpu.make_async_copy(
                            out_v.at[b],
                            out_hbm.at[pl.ds(0, 8), pl.ds(0, 1), :, :],
                            sem_s[b],
                        ).wait()

                    for g in range(8):
                        idx16 = idx_v[s, pl.ds(g * 16, 16)]

                        @plsc.parallel_loop(0, _D, unroll=8)
                        def _gather_d(d, idx16=idx16, g=g):
                            out_v[b, d // 8, 0, d % 8, pl.ds(g * 16, 16)] = (
                                plsc.load_gather(table_v, [idx16 + d * _V])
                            )
                    pltpu.async_copy(
                        out_v.at[b],
                        out_hbm.at[
                            pl.ds(pl.multiple_of(s * 8, 8), 8),
                            pl.ds(it, 1),
                            :,
                            :,
                        ],
                        sem_s[b],
                    )
                return carry

            lax.fori_loop(0, _S // 2, pair, 0)
            for b in range(2):
                pltpu.make_async_copy(
                    out_v.at[b],
                    out_hbm.at[pl.ds(0, 8), pl.ds(0, 1), :, :],
                    sem_s[b],
                ).wait()

    return k(tab_t_flat, idx_t)


def kernel(poi_counts, table):
    out4 = _sc_embedding_gather(table.T.reshape(-1), poi_counts.T)
    return (
        out4.reshape(_S, 8, _BATCH // 128, 8, 128)
        .transpose(2, 4, 0, 1, 3)
        .reshape(_BATCH, _S, _D)
    )
